# baseline (speedup 1.0000x reference)
import functools

import jax
import jax.numpy as jnp
from jax import lax
from jax.experimental import pallas as pl
from jax.experimental.pallas import tpu as pltpu
from jax.experimental.pallas import tpu_sc as plsc

N = 10000
E = 320000
D = 128
G = 64
NC = 2
NS = 16
L = 16

ROWS_PER_TILE = N // NS
DEG_CHUNK = 2000
DEG_PER_TILE = E // (NC * NS)
OUT_STRIDE = 624
OUT_LEN = 640
EDGE_C = 40
EDGES_PER_W = E // (NC * NS)
NB = 5

R = 2000
GRID = N // R


def _mesh():
    return plsc.VectorSubcoreMesh(
        core_axis_name="c", subcore_axis_name="s", num_cores=NC, num_subcores=NS
    )


def _dinv_body(dst_hbm, deg_hbm, hist, dbuf, shared, rbuf, obuf):
    cid = lax.axis_index("c")
    sid = lax.axis_index("s")

    zeros16 = jnp.zeros((L,), jnp.float32)
    ones16 = jnp.ones((L,), jnp.float32)

    def zbody(k, c):
        hist[pl.ds(k * L, L)] = zeros16
        return c

    lax.fori_loop(0, N // L, zbody, 0)

    base_t = (cid * NS + sid) * DEG_PER_TILE

    def obody(i, c):
        off = pl.multiple_of(base_t + i * DEG_CHUNK, 8)
        pltpu.sync_copy(dst_hbm.at[pl.ds(off, DEG_CHUNK)], dbuf)

        def ibody(k, c2):
            idx = dbuf[pl.ds(k * L, L)]
            plsc.addupdate_scatter(hist, [idx], ones16)
            return c2

        lax.fori_loop(0, DEG_CHUNK // L, ibody, 0)
        return c

    lax.fori_loop(0, DEG_PER_TILE // DEG_CHUNK, obody, 0)

    pltpu.sync_copy(hist, shared.at[sid])
    plsc.subcore_barrier()

    start = pl.multiple_of(sid * OUT_STRIDE, 8)
    for r in range(NS):
        pltpu.sync_copy(shared.at[r, pl.ds(start, OUT_LEN)], rbuf.at[r])

    def rbody(j, c):
        col = j * L
        deg = rbuf[0, pl.ds(col, L)]
        for r in range(1, NS):
            deg = deg + rbuf[r, pl.ds(col, L)]
        obuf[pl.ds(col, L)] = deg
        return c

    lax.fori_loop(0, OUT_LEN // L, rbody, 0)
    pltpu.sync_copy(obuf, deg_hbm.at[cid, pl.ds(start, OUT_LEN)])


_dinv_call = functools.partial(
    pl.kernel,
    out_type=jax.ShapeDtypeStruct((NC, N), jnp.float32),
    mesh=_mesh(),
    compiler_params=pltpu.CompilerParams(use_tc_tiling_on_sc=False, needs_layout_passes=False),
    scratch_types=[
        pltpu.VMEM((N,), jnp.float32),
        pltpu.VMEM((DEG_CHUNK,), jnp.int32),
        pltpu.VMEM_SHARED((NS, N), jnp.float32),
        pltpu.VMEM((NS, OUT_LEN), jnp.float32),
        pltpu.VMEM((OUT_LEN,), jnp.float32),
    ],
)(_dinv_body)


NCHUNK = EDGES_PER_W // EDGE_C
NGROUP = NCHUNK // NB


def _conv_body(y_hbm, src_hbm, dst_hbm, out_hbm, acc, sall, dall,
               rows, isem, gsems, ssems):
    cid = lax.axis_index("c")
    sid = lax.axis_index("s")
    r0 = sid * ROWS_PER_TILE
    init = pltpu.async_copy(y_hbm.at[pl.ds(r0, ROWS_PER_TILE)],
                            acc.at[pl.ds(r0, ROWS_PER_TILE)], isem)

    c0 = (cid * NS + sid) * NCHUNK
    pltpu.sync_copy(src_hbm.at[pl.ds(c0, NCHUNK)], sall)
    pltpu.sync_copy(dst_hbm.at[pl.ds(c0, NCHUNK)], dall)

    def gather(i, b):
        pltpu.async_copy(y_hbm.at[sall.at[i]], rows.at[b], gsems[b])

    def drain_gather(b):
        pltpu.make_async_copy(y_hbm.at[pl.ds(0, EDGE_C)], rows.at[b],
                              gsems[b]).wait()

    def scatter(i, b):
        pltpu.async_copy(rows.at[b], acc.at[dall.at[i]], ssems[b], add=True)

    def drain_scatter(b):
        pltpu.make_async_copy(y_hbm.at[pl.ds(0, EDGE_C)], rows.at[b],
                              ssems[b]).wait()

    for b in range(NB):
        gather(b, b)
    init.wait()
    plsc.subcore_barrier()

    def pbody(g, c):
        i0 = g * NB
        for b in range(NB):
            drain_gather(b)
            scatter(i0 + b, b)
        for b in range(NB):
            drain_scatter(b)

            @pl.when(g + 1 < NGROUP)
            def _():
                gather(i0 + NB + b, b)
        return c

    lax.fori_loop(0, NGROUP, pbody, 0)
    plsc.subcore_barrier()
    pltpu.sync_copy(acc.at[pl.ds(r0, ROWS_PER_TILE)], out_hbm.at[cid, pl.ds(r0, ROWS_PER_TILE)])


_conv_call = functools.partial(
    pl.kernel,
    out_type=jax.ShapeDtypeStruct((NC, N, D), jnp.float32),
    mesh=_mesh(),
    compiler_params=pltpu.CompilerParams(use_tc_tiling_on_sc=False, needs_layout_passes=False),
    scratch_types=[
        pltpu.VMEM_SHARED((N, D), jnp.float32),
        pltpu.VMEM((NCHUNK, EDGE_C), jnp.int32),
        pltpu.VMEM((NCHUNK, EDGE_C), jnp.int32),
        pltpu.VMEM((NB, EDGE_C, D), jnp.float32),
        pltpu.SemaphoreType.DMA,
        [pltpu.SemaphoreType.DMA] * NB,
        [pltpu.SemaphoreType.DMA] * NB,
    ],
)(_conv_body)


def _enc_kernel(x_ref, w1, b1, w2, b2, gw0, y1_ref):
    h = jnp.dot(x_ref[...], w1[...], preferred_element_type=jnp.float32) + b1[...]
    h = jnp.maximum(h, 0.0)
    h = jnp.dot(h, w2[...], preferred_element_type=jnp.float32) + b2[...]
    y1_ref[...] = jnp.dot(h, gw0[...], preferred_element_type=jnp.float32)


def _scale_kernel(u_ref, d0_ref, d1_ref, y_ref, dinv_ref):
    dv = jax.lax.rsqrt(d0_ref[...] + d1_ref[...] + 1.0)
    dinv_ref[...] = dv
    y_ref[...] = dv * u_ref[...]


def _mid_kernel(p0, p1, y1, dinv_ref, gb0, gw1, y2_ref):
    dv = dinv_ref[...]
    h = jnp.maximum(dv * (p0[...] + p1[...] - y1[...]) + gb0[...], 0.0)
    y2_ref[...] = dv * jnp.dot(h, gw1[...], preferred_element_type=jnp.float32)


def _fin_kernel(q0, q1, y2, dinv_ref, gb1, batch_ref, dw1, db1, dw2, db2,
                out_ref, pooled):
    i = pl.program_id(0)
    dv = dinv_ref[...]
    h = jnp.maximum(dv * (q0[...] + q1[...] - y2[...]) + gb1[...], 0.0)
    onehot = (batch_ref[...] == lax.broadcasted_iota(jnp.int32, (1, G), 1)
              ).astype(jnp.float32)
    contrib = lax.dot_general(onehot, h, (((0,), (0,)), ((), ())),
                              preferred_element_type=jnp.float32)

    @pl.when(i == 0)
    def _():
        pooled[...] = jnp.zeros_like(pooled)

    pooled[...] += contrib

    @pl.when(i == pl.num_programs(0) - 1)
    def _():
        t = jnp.maximum(
            jnp.dot(pooled[...], dw1[...], preferred_element_type=jnp.float32) + db1[...],
            0.0)
        out_ref[...] = jnp.dot(t, dw2[...], preferred_element_type=jnp.float32) + db2[...]


def _row_spec():
    return pl.BlockSpec((R, D), lambda i: (i, 0))


def _full_spec(shape):
    nd = len(shape)
    return pl.BlockSpec(shape, lambda i: (0,) * nd)


def _col_spec():
    return pl.BlockSpec((R, 1), lambda i: (i, 0))


def _enc_call(x, w1, b1, w2, b2, gw0):
    return pl.pallas_call(
        _enc_kernel,
        grid=(GRID,),
        in_specs=[_row_spec(), _full_spec((D, D)), _full_spec((1, D)),
                  _full_spec((D, D)), _full_spec((1, D)), _full_spec((D, D))],
        out_specs=_row_spec(),
        out_shape=jax.ShapeDtypeStruct((N, D), jnp.float32),
    )(x, w1, b1, w2, b2, gw0)


def _scale_call(u, deg0, deg1):
    return pl.pallas_call(
        _scale_kernel,
        grid=(GRID,),
        in_specs=[_row_spec(), _col_spec(), _col_spec()],
        out_specs=[_row_spec(), _col_spec()],
        out_shape=[jax.ShapeDtypeStruct((N, D), jnp.float32),
                   jax.ShapeDtypeStruct((N, 1), jnp.float32)],
    )(u, deg0, deg1)


def _mid_call(p0, p1, y1, dinv2, gb0, gw1):
    return pl.pallas_call(
        _mid_kernel,
        grid=(GRID,),
        in_specs=[_row_spec(), _row_spec(), _row_spec(), _col_spec(),
                  _full_spec((1, D)), _full_spec((D, D))],
        out_specs=_row_spec(),
        out_shape=jax.ShapeDtypeStruct((N, D), jnp.float32),
    )(p0, p1, y1, dinv2, gb0, gw1)


def _fin_call(q0, q1, y2, dinv2, gb1, batch2, dw1, db1, dw2, db2):
    return pl.pallas_call(
        _fin_kernel,
        grid=(GRID,),
        in_specs=[_row_spec(), _row_spec(), _row_spec(), _col_spec(),
                  _full_spec((1, D)),
                  pl.BlockSpec((R, 1), lambda i: (i, 0)),
                  _full_spec((D, D)), _full_spec((1, D)),
                  _full_spec((D, D)), _full_spec((1, D))],
        out_specs=_full_spec((G, D)),
        out_shape=jax.ShapeDtypeStruct((G, D), jnp.float32),
        scratch_shapes=[pltpu.VMEM((G, D), jnp.float32)],
    )(q0, q1, y2, dinv2, gb1, batch2, dw1, db1, dw2, db2)


@jax.jit
def kernel(x, edge_index, batch, enc_W1, enc_b1, enc_W2, enc_b2,
           gW0, gb0, gW1, gb1, dec_W1, dec_b1, dec_W2, dec_b2):
    src = edge_index[0]
    dst = edge_index[1]
    src2 = src.reshape(E // EDGE_C, EDGE_C)
    dst2 = dst.reshape(E // EDGE_C, EDGE_C)
    xw1 = _enc_call(x, enc_W1, enc_b1.reshape(1, D), enc_W2, enc_b2.reshape(1, D),
                    gW0)
    degp = _dinv_call(dst)
    y1, dinv2 = _scale_call(xw1, degp[0].reshape(N, 1), degp[1].reshape(N, 1))
    p = _conv_call(y1, src2, dst2)
    y2 = _mid_call(p[0], p[1], y1, dinv2, gb0.reshape(1, D), gW1)
    q = _conv_call(y2, src2, dst2)
    return _fin_call(q[0], q[1], y2, dinv2, gb1.reshape(1, D),
                     batch.reshape(N, 1), dec_W1, dec_b1.reshape(1, D),
                     dec_W2, dec_b2.reshape(1, D))


# speedup vs baseline: 1.0139x; 1.0139x over previous
import functools

import jax
import jax.numpy as jnp
from jax import lax
from jax.experimental import pallas as pl
from jax.experimental.pallas import tpu as pltpu
from jax.experimental.pallas import tpu_sc as plsc

N = 10000
E = 320000
D = 128
G = 64
NC = 2
NS = 16
L = 16

ROWS_PER_TILE = N // NS
DEG_CHUNK = 2000
DEG_PER_TILE = E // (NC * NS)
OUT_STRIDE = 624
OUT_LEN = 640
EDGE_C = 40
EDGES_PER_W = E // (NC * NS)
NB = 6

R = 2000
GRID = N // R


def _mesh():
    return plsc.VectorSubcoreMesh(
        core_axis_name="c", subcore_axis_name="s", num_cores=NC, num_subcores=NS
    )


def _dinv_body(dst_hbm, deg_hbm, hist, dbuf, shared, rbuf, obuf):
    cid = lax.axis_index("c")
    sid = lax.axis_index("s")

    zeros16 = jnp.zeros((L,), jnp.float32)
    ones16 = jnp.ones((L,), jnp.float32)

    def zbody(k, c):
        hist[pl.ds(k * L, L)] = zeros16
        return c

    lax.fori_loop(0, N // L, zbody, 0)

    base_t = (cid * NS + sid) * DEG_PER_TILE

    def obody(i, c):
        off = pl.multiple_of(base_t + i * DEG_CHUNK, 8)
        pltpu.sync_copy(dst_hbm.at[pl.ds(off, DEG_CHUNK)], dbuf)

        def ibody(k, c2):
            idx = dbuf[pl.ds(k * L, L)]
            plsc.addupdate_scatter(hist, [idx], ones16)
            return c2

        lax.fori_loop(0, DEG_CHUNK // L, ibody, 0)
        return c

    lax.fori_loop(0, DEG_PER_TILE // DEG_CHUNK, obody, 0)

    pltpu.sync_copy(hist, shared.at[sid])
    plsc.subcore_barrier()

    start = pl.multiple_of(sid * OUT_STRIDE, 8)
    for r in range(NS):
        pltpu.sync_copy(shared.at[r, pl.ds(start, OUT_LEN)], rbuf.at[r])

    def rbody(j, c):
        col = j * L
        deg = rbuf[0, pl.ds(col, L)]
        for r in range(1, NS):
            deg = deg + rbuf[r, pl.ds(col, L)]
        obuf[pl.ds(col, L)] = deg
        return c

    lax.fori_loop(0, OUT_LEN // L, rbody, 0)
    pltpu.sync_copy(obuf, deg_hbm.at[cid, pl.ds(start, OUT_LEN)])


_dinv_call = functools.partial(
    pl.kernel,
    out_type=jax.ShapeDtypeStruct((NC, N), jnp.float32),
    mesh=_mesh(),
    compiler_params=pltpu.CompilerParams(use_tc_tiling_on_sc=False, needs_layout_passes=False),
    scratch_types=[
        pltpu.VMEM((N,), jnp.float32),
        pltpu.VMEM((DEG_CHUNK,), jnp.int32),
        pltpu.VMEM_SHARED((NS, N), jnp.float32),
        pltpu.VMEM((NS, OUT_LEN), jnp.float32),
        pltpu.VMEM((OUT_LEN,), jnp.float32),
    ],
)(_dinv_body)


NCHUNK = EDGES_PER_W // EDGE_C
NGROUP = NCHUNK // NB
NTAIL = NCHUNK - NGROUP * NB


def _conv_body(y_hbm, src_hbm, dst_hbm, out_hbm, acc, sall, dall,
               rows, isem, gsems, ssems):
    cid = lax.axis_index("c")
    sid = lax.axis_index("s")
    r0 = sid * ROWS_PER_TILE
    init = pltpu.async_copy(y_hbm.at[pl.ds(r0, ROWS_PER_TILE)],
                            acc.at[pl.ds(r0, ROWS_PER_TILE)], isem)

    c0 = (cid * NS + sid) * NCHUNK
    pltpu.sync_copy(src_hbm.at[pl.ds(c0, NCHUNK)], sall)
    pltpu.sync_copy(dst_hbm.at[pl.ds(c0, NCHUNK)], dall)

    def gather(i, b):
        pltpu.async_copy(y_hbm.at[sall.at[i]], rows.at[b], gsems[b])

    def drain_gather(b):
        pltpu.make_async_copy(y_hbm.at[pl.ds(0, EDGE_C)], rows.at[b],
                              gsems[b]).wait()

    def scatter(i, b):
        pltpu.async_copy(rows.at[b], acc.at[dall.at[i]], ssems[b], add=True)

    def drain_scatter(b):
        pltpu.make_async_copy(y_hbm.at[pl.ds(0, EDGE_C)], rows.at[b],
                              ssems[b]).wait()

    for b in range(NB):
        gather(b, b)
    init.wait()
    plsc.subcore_barrier()

    def pbody(g, c):
        i0 = g * NB
        for b in range(NB):
            drain_gather(b)
            scatter(i0 + b, b)
        for b in range(NB):
            drain_scatter(b)

            @pl.when(i0 + NB + b < NCHUNK)
            def _():
                gather(i0 + NB + b, b)
        return c

    lax.fori_loop(0, NGROUP, pbody, 0)
    for b in range(NTAIL):
        drain_gather(b)
        scatter(NGROUP * NB + b, b)
    for b in range(NTAIL):
        drain_scatter(b)
    plsc.subcore_barrier()
    pltpu.sync_copy(acc.at[pl.ds(r0, ROWS_PER_TILE)], out_hbm.at[cid, pl.ds(r0, ROWS_PER_TILE)])


_conv_call = functools.partial(
    pl.kernel,
    out_type=jax.ShapeDtypeStruct((NC, N, D), jnp.float32),
    mesh=_mesh(),
    compiler_params=pltpu.CompilerParams(use_tc_tiling_on_sc=False, needs_layout_passes=False),
    scratch_types=[
        pltpu.VMEM_SHARED((N, D), jnp.float32),
        pltpu.VMEM((NCHUNK, EDGE_C), jnp.int32),
        pltpu.VMEM((NCHUNK, EDGE_C), jnp.int32),
        pltpu.VMEM((NB, EDGE_C, D), jnp.float32),
        pltpu.SemaphoreType.DMA,
        [pltpu.SemaphoreType.DMA] * NB,
        [pltpu.SemaphoreType.DMA] * NB,
    ],
)(_conv_body)


def _enc_kernel(x_ref, w1, b1, w2, b2, gw0, y1_ref):
    h = jnp.dot(x_ref[...], w1[...], preferred_element_type=jnp.float32) + b1[...]
    h = jnp.maximum(h, 0.0)
    h = jnp.dot(h, w2[...], preferred_element_type=jnp.float32) + b2[...]
    y1_ref[...] = jnp.dot(h, gw0[...], preferred_element_type=jnp.float32)


def _scale_kernel(u_ref, d0_ref, d1_ref, y_ref, dinv_ref):
    dv = jax.lax.rsqrt(d0_ref[...] + d1_ref[...] + 1.0)
    dinv_ref[...] = dv
    y_ref[...] = dv * u_ref[...]


def _mid_kernel(p0, p1, y1, dinv_ref, gb0, gw1, y2_ref):
    dv = dinv_ref[...]
    h = jnp.maximum(dv * (p0[...] + p1[...] - y1[...]) + gb0[...], 0.0)
    y2_ref[...] = dv * jnp.dot(h, gw1[...], preferred_element_type=jnp.float32)


def _fin_kernel(q0, q1, y2, dinv_ref, gb1, batch_ref, dw1, db1, dw2, db2,
                out_ref, pooled):
    i = pl.program_id(0)
    dv = dinv_ref[...]
    h = jnp.maximum(dv * (q0[...] + q1[...] - y2[...]) + gb1[...], 0.0)
    onehot = (batch_ref[...] == lax.broadcasted_iota(jnp.int32, (1, G), 1)
              ).astype(jnp.float32)
    contrib = lax.dot_general(onehot, h, (((0,), (0,)), ((), ())),
                              preferred_element_type=jnp.float32)

    @pl.when(i == 0)
    def _():
        pooled[...] = jnp.zeros_like(pooled)

    pooled[...] += contrib

    @pl.when(i == pl.num_programs(0) - 1)
    def _():
        t = jnp.maximum(
            jnp.dot(pooled[...], dw1[...], preferred_element_type=jnp.float32) + db1[...],
            0.0)
        out_ref[...] = jnp.dot(t, dw2[...], preferred_element_type=jnp.float32) + db2[...]


def _row_spec():
    return pl.BlockSpec((R, D), lambda i: (i, 0))


def _full_spec(shape):
    nd = len(shape)
    return pl.BlockSpec(shape, lambda i: (0,) * nd)


def _col_spec():
    return pl.BlockSpec((R, 1), lambda i: (i, 0))


def _enc_call(x, w1, b1, w2, b2, gw0):
    return pl.pallas_call(
        _enc_kernel,
        grid=(GRID,),
        in_specs=[_row_spec(), _full_spec((D, D)), _full_spec((1, D)),
                  _full_spec((D, D)), _full_spec((1, D)), _full_spec((D, D))],
        out_specs=_row_spec(),
        out_shape=jax.ShapeDtypeStruct((N, D), jnp.float32),
    )(x, w1, b1, w2, b2, gw0)


def _scale_call(u, deg0, deg1):
    return pl.pallas_call(
        _scale_kernel,
        grid=(GRID,),
        in_specs=[_row_spec(), _col_spec(), _col_spec()],
        out_specs=[_row_spec(), _col_spec()],
        out_shape=[jax.ShapeDtypeStruct((N, D), jnp.float32),
                   jax.ShapeDtypeStruct((N, 1), jnp.float32)],
    )(u, deg0, deg1)


def _mid_call(p0, p1, y1, dinv2, gb0, gw1):
    return pl.pallas_call(
        _mid_kernel,
        grid=(GRID,),
        in_specs=[_row_spec(), _row_spec(), _row_spec(), _col_spec(),
                  _full_spec((1, D)), _full_spec((D, D))],
        out_specs=_row_spec(),
        out_shape=jax.ShapeDtypeStruct((N, D), jnp.float32),
    )(p0, p1, y1, dinv2, gb0, gw1)


def _fin_call(q0, q1, y2, dinv2, gb1, batch2, dw1, db1, dw2, db2):
    return pl.pallas_call(
        _fin_kernel,
        grid=(GRID,),
        in_specs=[_row_spec(), _row_spec(), _row_spec(), _col_spec(),
                  _full_spec((1, D)),
                  pl.BlockSpec((R, 1), lambda i: (i, 0)),
                  _full_spec((D, D)), _full_spec((1, D)),
                  _full_spec((D, D)), _full_spec((1, D))],
        out_specs=_full_spec((G, D)),
        out_shape=jax.ShapeDtypeStruct((G, D), jnp.float32),
        scratch_shapes=[pltpu.VMEM((G, D), jnp.float32)],
    )(q0, q1, y2, dinv2, gb1, batch2, dw1, db1, dw2, db2)


@jax.jit
def kernel(x, edge_index, batch, enc_W1, enc_b1, enc_W2, enc_b2,
           gW0, gb0, gW1, gb1, dec_W1, dec_b1, dec_W2, dec_b2):
    src = edge_index[0]
    dst = edge_index[1]
    src2 = src.reshape(E // EDGE_C, EDGE_C)
    dst2 = dst.reshape(E // EDGE_C, EDGE_C)
    xw1 = _enc_call(x, enc_W1, enc_b1.reshape(1, D), enc_W2, enc_b2.reshape(1, D),
                    gW0)
    degp = _dinv_call(dst)
    y1, dinv2 = _scale_call(xw1, degp[0].reshape(N, 1), degp[1].reshape(N, 1))
    p = _conv_call(y1, src2, dst2)
    y2 = _mid_call(p[0], p[1], y1, dinv2, gb0.reshape(1, D), gW1)
    q = _conv_call(y2, src2, dst2)
    return _fin_call(q[0], q[1], y2, dinv2, gb1.reshape(1, D),
                     batch.reshape(N, 1), dec_W1, dec_b1.reshape(1, D),
                     dec_W2, dec_b2.reshape(1, D))


# speedup vs baseline: 1.0260x; 1.0120x over previous
import functools

import jax
import jax.numpy as jnp
from jax import lax
from jax.experimental import pallas as pl
from jax.experimental.pallas import tpu as pltpu
from jax.experimental.pallas import tpu_sc as plsc

N = 10000
E = 320000
D = 128
G = 64
NC = 2
NS = 16
L = 16

ROWS_PER_TILE = N // NS
DEG_CHUNK = 2000
DEG_PER_TILE = E // (NC * NS)
OUT_STRIDE = 624
OUT_LEN = 640
EDGE_C = 40
EDGES_PER_W = E // (NC * NS)
NB = 7

R = 2000
GRID = N // R


def _mesh():
    return plsc.VectorSubcoreMesh(
        core_axis_name="c", subcore_axis_name="s", num_cores=NC, num_subcores=NS
    )


def _dinv_body(dst_hbm, deg_hbm, hist, dbuf, shared, rbuf, obuf):
    cid = lax.axis_index("c")
    sid = lax.axis_index("s")

    zeros16 = jnp.zeros((L,), jnp.float32)
    ones16 = jnp.ones((L,), jnp.float32)

    def zbody(k, c):
        hist[pl.ds(k * L, L)] = zeros16
        return c

    lax.fori_loop(0, N // L, zbody, 0)

    base_t = (cid * NS + sid) * DEG_PER_TILE

    def obody(i, c):
        off = pl.multiple_of(base_t + i * DEG_CHUNK, 8)
        pltpu.sync_copy(dst_hbm.at[pl.ds(off, DEG_CHUNK)], dbuf)

        def ibody(k, c2):
            idx = dbuf[pl.ds(k * L, L)]
            plsc.addupdate_scatter(hist, [idx], ones16)
            return c2

        lax.fori_loop(0, DEG_CHUNK // L, ibody, 0)
        return c

    lax.fori_loop(0, DEG_PER_TILE // DEG_CHUNK, obody, 0)

    pltpu.sync_copy(hist, shared.at[sid])
    plsc.subcore_barrier()

    start = pl.multiple_of(sid * OUT_STRIDE, 8)
    for r in range(NS):
        pltpu.sync_copy(shared.at[r, pl.ds(start, OUT_LEN)], rbuf.at[r])

    def rbody(j, c):
        col = j * L
        deg = rbuf[0, pl.ds(col, L)]
        for r in range(1, NS):
            deg = deg + rbuf[r, pl.ds(col, L)]
        obuf[pl.ds(col, L)] = deg
        return c

    lax.fori_loop(0, OUT_LEN // L, rbody, 0)
    pltpu.sync_copy(obuf, deg_hbm.at[cid, pl.ds(start, OUT_LEN)])


_dinv_call = functools.partial(
    pl.kernel,
    out_type=jax.ShapeDtypeStruct((NC, N), jnp.float32),
    mesh=_mesh(),
    compiler_params=pltpu.CompilerParams(use_tc_tiling_on_sc=False, needs_layout_passes=False),
    scratch_types=[
        pltpu.VMEM((N,), jnp.float32),
        pltpu.VMEM((DEG_CHUNK,), jnp.int32),
        pltpu.VMEM_SHARED((NS, N), jnp.float32),
        pltpu.VMEM((NS, OUT_LEN), jnp.float32),
        pltpu.VMEM((OUT_LEN,), jnp.float32),
    ],
)(_dinv_body)


NCHUNK = EDGES_PER_W // EDGE_C
NGROUP = NCHUNK // NB
NTAIL = NCHUNK - NGROUP * NB


def _conv_body(y_hbm, src_hbm, dst_hbm, out_hbm, acc, sall, dbuf,
               rows, isem, dsem, gsems, ssems):
    cid = lax.axis_index("c")
    sid = lax.axis_index("s")
    r0 = sid * ROWS_PER_TILE
    init = pltpu.async_copy(y_hbm.at[pl.ds(r0, ROWS_PER_TILE)],
                            acc.at[pl.ds(r0, ROWS_PER_TILE)], isem)

    c0 = (cid * NS + sid) * NCHUNK
    pltpu.sync_copy(src_hbm.at[pl.ds(c0, NCHUNK)], sall)
    pltpu.async_copy(dst_hbm.at[pl.ds(c0, NB)], dbuf.at[0], dsem)

    def gather(i, b):
        pltpu.async_copy(y_hbm.at[sall.at[i]], rows.at[b], gsems[b])

    def drain_gather(b):
        pltpu.make_async_copy(y_hbm.at[pl.ds(0, EDGE_C)], rows.at[b],
                              gsems[b]).wait()

    def scatter(par, b):
        pltpu.async_copy(rows.at[b], acc.at[dbuf.at[par, b]], ssems[b], add=True)

    def drain_scatter(b):
        pltpu.make_async_copy(y_hbm.at[pl.ds(0, EDGE_C)], rows.at[b],
                              ssems[b]).wait()

    for b in range(NB):
        gather(b, b)
    init.wait()
    plsc.subcore_barrier()

    def pbody(g, c):
        i0 = g * NB
        parity = lax.rem(g, 2)
        pltpu.make_async_copy(dst_hbm.at[pl.ds(0, NB)], dbuf.at[0], dsem).wait()

        @pl.when(g + 2 <= NGROUP)
        def _():
            pltpu.async_copy(dst_hbm.at[pl.ds(c0 + (g + 1) * NB, NB)],
                             dbuf.at[1 - parity], dsem)

        if NTAIL > 0:
            @pl.when(g + 1 == NGROUP)
            def _():
                pltpu.async_copy(dst_hbm.at[pl.ds(c0 + NGROUP * NB, NTAIL)],
                                 dbuf.at[1 - parity, pl.ds(0, NTAIL)], dsem)

        for b in range(NB):
            drain_gather(b)
            scatter(parity, b)
        for b in range(NB):
            drain_scatter(b)

            @pl.when(i0 + NB + b < NCHUNK)
            def _():
                gather(i0 + NB + b, b)
        return c

    lax.fori_loop(0, NGROUP, pbody, 0)
    if NTAIL > 0:
        tail_par = NGROUP % 2
        pltpu.make_async_copy(dst_hbm.at[pl.ds(0, NTAIL)],
                              dbuf.at[0, pl.ds(0, NTAIL)], dsem).wait()
        for b in range(NTAIL):
            drain_gather(b)
            scatter(tail_par, b)
        for b in range(NTAIL):
            drain_scatter(b)
    plsc.subcore_barrier()
    pltpu.sync_copy(acc.at[pl.ds(r0, ROWS_PER_TILE)], out_hbm.at[cid, pl.ds(r0, ROWS_PER_TILE)])


_conv_call = functools.partial(
    pl.kernel,
    out_type=jax.ShapeDtypeStruct((NC, N, D), jnp.float32),
    mesh=_mesh(),
    compiler_params=pltpu.CompilerParams(use_tc_tiling_on_sc=False, needs_layout_passes=False),
    scratch_types=[
        pltpu.VMEM_SHARED((N, D), jnp.float32),
        pltpu.VMEM((NCHUNK, EDGE_C), jnp.int32),
        pltpu.VMEM((2, NB, EDGE_C), jnp.int32),
        pltpu.VMEM((NB, EDGE_C, D), jnp.float32),
        pltpu.SemaphoreType.DMA,
        pltpu.SemaphoreType.DMA,
        [pltpu.SemaphoreType.DMA] * NB,
        [pltpu.SemaphoreType.DMA] * NB,
    ],
)(_conv_body)


def _enc_kernel(x_ref, w1, b1, w2, b2, gw0, y1_ref):
    h = jnp.dot(x_ref[...], w1[...], preferred_element_type=jnp.float32) + b1[...]
    h = jnp.maximum(h, 0.0)
    h = jnp.dot(h, w2[...], preferred_element_type=jnp.float32) + b2[...]
    y1_ref[...] = jnp.dot(h, gw0[...], preferred_element_type=jnp.float32)


def _scale_kernel(u_ref, d0_ref, d1_ref, y_ref, dinv_ref):
    dv = jax.lax.rsqrt(d0_ref[...] + d1_ref[...] + 1.0)
    dinv_ref[...] = dv
    y_ref[...] = dv * u_ref[...]


def _mid_kernel(p0, p1, y1, dinv_ref, gb0, gw1, y2_ref):
    dv = dinv_ref[...]
    h = jnp.maximum(dv * (p0[...] + p1[...] - y1[...]) + gb0[...], 0.0)
    y2_ref[...] = dv * jnp.dot(h, gw1[...], preferred_element_type=jnp.float32)


def _fin_kernel(q0, q1, y2, dinv_ref, gb1, batch_ref, dw1, db1, dw2, db2,
                out_ref, pooled):
    i = pl.program_id(0)
    dv = dinv_ref[...]
    h = jnp.maximum(dv * (q0[...] + q1[...] - y2[...]) + gb1[...], 0.0)
    onehot = (batch_ref[...] == lax.broadcasted_iota(jnp.int32, (1, G), 1)
              ).astype(jnp.float32)
    contrib = lax.dot_general(onehot, h, (((0,), (0,)), ((), ())),
                              preferred_element_type=jnp.float32)

    @pl.when(i == 0)
    def _():
        pooled[...] = jnp.zeros_like(pooled)

    pooled[...] += contrib

    @pl.when(i == pl.num_programs(0) - 1)
    def _():
        t = jnp.maximum(
            jnp.dot(pooled[...], dw1[...], preferred_element_type=jnp.float32) + db1[...],
            0.0)
        out_ref[...] = jnp.dot(t, dw2[...], preferred_element_type=jnp.float32) + db2[...]


def _row_spec():
    return pl.BlockSpec((R, D), lambda i: (i, 0))


def _full_spec(shape):
    nd = len(shape)
    return pl.BlockSpec(shape, lambda i: (0,) * nd)


def _col_spec():
    return pl.BlockSpec((R, 1), lambda i: (i, 0))


def _enc_call(x, w1, b1, w2, b2, gw0):
    return pl.pallas_call(
        _enc_kernel,
        grid=(GRID,),
        in_specs=[_row_spec(), _full_spec((D, D)), _full_spec((1, D)),
                  _full_spec((D, D)), _full_spec((1, D)), _full_spec((D, D))],
        out_specs=_row_spec(),
        out_shape=jax.ShapeDtypeStruct((N, D), jnp.float32),
    )(x, w1, b1, w2, b2, gw0)


def _scale_call(u, deg0, deg1):
    return pl.pallas_call(
        _scale_kernel,
        grid=(GRID,),
        in_specs=[_row_spec(), _col_spec(), _col_spec()],
        out_specs=[_row_spec(), _col_spec()],
        out_shape=[jax.ShapeDtypeStruct((N, D), jnp.float32),
                   jax.ShapeDtypeStruct((N, 1), jnp.float32)],
    )(u, deg0, deg1)


def _mid_call(p0, p1, y1, dinv2, gb0, gw1):
    return pl.pallas_call(
        _mid_kernel,
        grid=(GRID,),
        in_specs=[_row_spec(), _row_spec(), _row_spec(), _col_spec(),
                  _full_spec((1, D)), _full_spec((D, D))],
        out_specs=_row_spec(),
        out_shape=jax.ShapeDtypeStruct((N, D), jnp.float32),
    )(p0, p1, y1, dinv2, gb0, gw1)


def _fin_call(q0, q1, y2, dinv2, gb1, batch2, dw1, db1, dw2, db2):
    return pl.pallas_call(
        _fin_kernel,
        grid=(GRID,),
        in_specs=[_row_spec(), _row_spec(), _row_spec(), _col_spec(),
                  _full_spec((1, D)),
                  pl.BlockSpec((R, 1), lambda i: (i, 0)),
                  _full_spec((D, D)), _full_spec((1, D)),
                  _full_spec((D, D)), _full_spec((1, D))],
        out_specs=_full_spec((G, D)),
        out_shape=jax.ShapeDtypeStruct((G, D), jnp.float32),
        scratch_shapes=[pltpu.VMEM((G, D), jnp.float32)],
    )(q0, q1, y2, dinv2, gb1, batch2, dw1, db1, dw2, db2)


@jax.jit
def kernel(x, edge_index, batch, enc_W1, enc_b1, enc_W2, enc_b2,
           gW0, gb0, gW1, gb1, dec_W1, dec_b1, dec_W2, dec_b2):
    src = edge_index[0]
    dst = edge_index[1]
    src2 = src.reshape(E // EDGE_C, EDGE_C)
    dst2 = dst.reshape(E // EDGE_C, EDGE_C)
    xw1 = _enc_call(x, enc_W1, enc_b1.reshape(1, D), enc_W2, enc_b2.reshape(1, D),
                    gW0)
    degp = _dinv_call(dst)
    y1, dinv2 = _scale_call(xw1, degp[0].reshape(N, 1), degp[1].reshape(N, 1))
    p = _conv_call(y1, src2, dst2)
    y2 = _mid_call(p[0], p[1], y1, dinv2, gb0.reshape(1, D), gW1)
    q = _conv_call(y2, src2, dst2)
    return _fin_call(q[0], q[1], y2, dinv2, gb1.reshape(1, D),
                     batch.reshape(N, 1), dec_W1, dec_b1.reshape(1, D),
                     dec_W2, dec_b2.reshape(1, D))
